# baseline (speedup 1.0000x reference)
import functools

import jax
import jax.numpy as jnp
import numpy as np
from jax import lax
from jax.experimental import pallas as pl
from jax.experimental.pallas import tpu as pltpu

try:
    from jax import shard_map as _shard_map_fn
except ImportError:
    from jax.experimental.shard_map import shard_map as _shard_map_fn

P = jax.sharding.PartitionSpec

_BN_EPS = 1e-5
_VMEM_LIMIT = 48 * 1024 * 1024
_TILE_STATS = 16
_TILE_APPLY = 8


def _silu(h):
    m = 0.5 * h
    return m + m * jnp.tanh(m)


def _lane_masks(length):
    lane = lax.broadcasted_iota(jnp.int32, (1, length), 1)
    return lane == 0, lane == length - 1


def _conv3(v, w_ref, first_mask, last_mask):
    _, ell = v.shape
    y = jnp.dot(w_ref[1], v.astype(jnp.bfloat16),
                preferred_element_type=jnp.float32)
    vm1 = jnp.where(first_mask, 0.0, pltpu.roll(v, shift=1, axis=1))
    y = y + jnp.dot(w_ref[0], vm1.astype(jnp.bfloat16),
                    preferred_element_type=jnp.float32)
    vp1 = jnp.where(last_mask, 0.0, pltpu.roll(v, shift=ell - 1, axis=1))
    y = y + jnp.dot(w_ref[2], vp1.astype(jnp.bfloat16),
                    preferred_element_type=jnp.float32)
    return y


def _stats_kernel(x_ref, w1_ref, b1_ref, s_ref):
    tn, _, ell = x_ref.shape
    co = s_ref.shape[1]
    first, last = _lane_masks(ell)
    acc_s = jnp.zeros((co, 1), jnp.float32)
    acc_q = jnp.zeros((co, 1), jnp.float32)
    for n in range(tn):
        h = _silu(_conv3(x_ref[n], w1_ref, first, last) + b1_ref[...])
        acc_s = acc_s + jnp.sum(h, axis=1, keepdims=True)
        acc_q = acc_q + jnp.sum(h * h, axis=1, keepdims=True)
    s_ref[0, :, 0:1] = acc_s
    s_ref[0, :, 1:2] = acc_q


def _apply_kernel(has_proj, x_ref, sc_ref, sh_ref, w1_ref, b1_ref, w2_ref,
                  b2_ref, *rest):
    if has_proj:
        wp_ref, o_ref = rest
    else:
        (o_ref,) = rest
    tn, _, ell = x_ref.shape
    first, last = _lane_masks(ell)
    scale = sc_ref[...]
    shift = sh_ref[...]
    for n in range(tn):
        x_n = x_ref[n]
        h = _silu(_conv3(x_n, w1_ref, first, last) + b1_ref[...])
        h = h * scale + shift
        y = _conv3(h, w2_ref, first, last) + b2_ref[...]
        if has_proj:
            y = y + jnp.dot(wp_ref[...], x_n.astype(jnp.bfloat16),
                            preferred_element_type=jnp.float32)
        else:
            y = y + x_n
        o_ref[n] = y


def _conv_weight(w):
    return jnp.transpose(w, (2, 0, 1)).astype(jnp.bfloat16)


def _const_spec(shape):
    rank = len(shape)
    return pl.BlockSpec(shape, lambda *_, _r=rank: (0,) * _r)


def _tile(n, want):
    tn = min(n, want)
    while n % tn:
        tn -= 1
    return tn


def _stats_pass(x, w1_k, b1_2):
    n, ci, ell = x.shape
    co = b1_2.shape[0]
    tn1 = _tile(n, _TILE_STATS)
    g1 = n // tn1
    stats = pl.pallas_call(
        _stats_kernel,
        out_shape=jax.ShapeDtypeStruct((g1, co, 128), jnp.float32),
        grid=(g1,),
        in_specs=[pl.BlockSpec((tn1, ci, ell), lambda i: (i, 0, 0)),
                  _const_spec(w1_k.shape), _const_spec(b1_2.shape)],
        out_specs=pl.BlockSpec((1, co, 128), lambda i: (i, 0, 0)),
        compiler_params=pltpu.CompilerParams(
            dimension_semantics=("arbitrary",), vmem_limit_bytes=_VMEM_LIMIT),
    )(x, w1_k, b1_2)
    loc_sum = jnp.sum(stats[:, :, 0], axis=0).reshape(co, 1)
    loc_sq = jnp.sum(stats[:, :, 1], axis=0).reshape(co, 1)
    return loc_sum, loc_sq


def _apply_pass(x, scale, shift, w1_k, b1_2, w2_k, b2_2, wp_2):
    n, ci, ell = x.shape
    co = b1_2.shape[0]
    has_proj = wp_2 is not None
    tn2 = _tile(n, _TILE_APPLY)
    g2 = n // tn2
    x_spec = pl.BlockSpec((tn2, ci, ell), lambda i: (i, 0, 0))
    ins = [x, scale, shift, w1_k, b1_2, w2_k, b2_2]
    in_specs = [x_spec, _const_spec(scale.shape), _const_spec(shift.shape),
                _const_spec(w1_k.shape), _const_spec(b1_2.shape),
                _const_spec(w2_k.shape), _const_spec(b2_2.shape)]
    if has_proj:
        ins.append(wp_2)
        in_specs.append(_const_spec(wp_2.shape))
    return pl.pallas_call(
        functools.partial(_apply_kernel, has_proj),
        out_shape=jax.ShapeDtypeStruct((n, co, ell), jnp.float32),
        grid=(g2,),
        in_specs=in_specs,
        out_specs=pl.BlockSpec((tn2, co, ell), lambda i: (i, 0, 0)),
        compiler_params=pltpu.CompilerParams(
            dimension_semantics=("arbitrary",), vmem_limit_bytes=_VMEM_LIMIT),
    )(*ins)


def kernel(x, w1, b1, gamma, beta, w2, b2, wp, bp):
    n, ci, ell = x.shape
    co = w1.shape[0]
    has_proj = wp is not None

    w1_k = _conv_weight(w1)
    w2_k = _conv_weight(w2)
    b1_2 = b1.reshape(co, 1)
    b2_2 = (b2 + (bp if has_proj else 0.0)).reshape(co, 1)
    g_2 = gamma.reshape(co, 1)
    bt_2 = beta.reshape(co, 1)
    wp_2 = wp[:, :, 0].astype(jnp.bfloat16) if has_proj else None
    inv_count = 1.0 / float(n * ell)

    devs = jax.devices()
    n_dev = 2 if (len(devs) >= 2 and n % 2 == 0) else 1

    if n_dev == 1:
        loc_sum, loc_sq = _stats_pass(x, w1_k, b1_2)
        mean = loc_sum * inv_count
        var = loc_sq * inv_count - mean * mean
        scale = g_2 * lax.rsqrt(var + _BN_EPS)
        shift = bt_2 - mean * scale
        return _apply_pass(x, scale, shift, w1_k, b1_2, w2_k, b2_2, wp_2)

    mesh = jax.sharding.Mesh(np.asarray(devs[:2]), ("b",))

    def _sharded(x_sh, w1_k, b1_2, w2_k, b2_2, g_2, bt_2, *maybe_wp):
        wp_sh = maybe_wp[0] if maybe_wp else None
        loc_sum, loc_sq = _stats_pass(x_sh, w1_k, b1_2)
        tot_sum = lax.psum(loc_sum, "b")
        tot_sq = lax.psum(loc_sq, "b")
        mean = tot_sum * inv_count
        var = tot_sq * inv_count - mean * mean
        scale = g_2 * lax.rsqrt(var + _BN_EPS)
        shift = bt_2 - mean * scale
        return _apply_pass(x_sh, scale, shift, w1_k, b1_2, w2_k, b2_2, wp_sh)

    args = [x, w1_k, b1_2, w2_k, b2_2, g_2, bt_2]
    in_specs = [P("b")] + [P()] * 6
    if has_proj:
        args.append(wp_2)
        in_specs.append(P())
    run = _shard_map_fn(_sharded, mesh=mesh, in_specs=tuple(in_specs),
                        out_specs=P("b"), check_vma=False)
    return run(*args)


# speedup vs baseline: 2.2312x; 2.2312x over previous
import functools

import jax
import jax.numpy as jnp
from jax import lax
from jax.experimental import pallas as pl
from jax.experimental.pallas import tpu as pltpu

_BN_EPS = 1e-5
_VMEM_LIMIT = 48 * 1024 * 1024
_TILE_STATS = 32
_TILE_APPLY = 16


def _silu(h):
    m = 0.5 * h
    return m + m * jnp.tanh(m)


def _lane_masks(length):
    lane = lax.broadcasted_iota(jnp.int32, (1, length), 1)
    return lane == 0, lane == length - 1


def _conv3(v, w_ref, first_mask, last_mask):
    _, ell = v.shape
    y = jnp.dot(w_ref[1], v.astype(jnp.bfloat16),
                preferred_element_type=jnp.float32)
    vm1 = jnp.where(first_mask, 0.0, pltpu.roll(v, shift=1, axis=1))
    y = y + jnp.dot(w_ref[0], vm1.astype(jnp.bfloat16),
                    preferred_element_type=jnp.float32)
    vp1 = jnp.where(last_mask, 0.0, pltpu.roll(v, shift=ell - 1, axis=1))
    y = y + jnp.dot(w_ref[2], vp1.astype(jnp.bfloat16),
                    preferred_element_type=jnp.float32)
    return y


def _stats_kernel(x_ref, w1_ref, b1_ref, s_ref):
    tn, _, ell = x_ref.shape
    co = s_ref.shape[1]
    first, last = _lane_masks(ell)
    acc_s = jnp.zeros((co, 1), jnp.float32)
    acc_q = jnp.zeros((co, 1), jnp.float32)
    for n in range(tn):
        h = _silu(_conv3(x_ref[n], w1_ref, first, last) + b1_ref[...])
        acc_s = acc_s + jnp.sum(h, axis=1, keepdims=True)
        acc_q = acc_q + jnp.sum(h * h, axis=1, keepdims=True)
    s_ref[0, :, 0:1] = acc_s
    s_ref[0, :, 1:2] = acc_q


def _apply_kernel(has_proj, x_ref, sc_ref, sh_ref, w1_ref, b1_ref, w2_ref,
                  b2_ref, *rest):
    if has_proj:
        wp_ref, o_ref = rest
    else:
        (o_ref,) = rest
    tn, _, ell = x_ref.shape
    first, last = _lane_masks(ell)
    scale = sc_ref[...]
    shift = sh_ref[...]
    for n in range(tn):
        x_n = x_ref[n]
        xb = x_n.astype(jnp.bfloat16)
        if has_proj:
            p = jnp.dot(wp_ref[...], xb, preferred_element_type=jnp.float32)
        h = _silu(_conv3(x_n, w1_ref, first, last) + b1_ref[...])
        h = h * scale + shift
        y = _conv3(h, w2_ref, first, last) + b2_ref[...]
        if has_proj:
            y = y + p
        else:
            y = y + x_n
        o_ref[n] = y


def _conv_weight(w):
    return jnp.transpose(w, (2, 0, 1)).astype(jnp.bfloat16)


def _const_spec(shape):
    rank = len(shape)
    return pl.BlockSpec(shape, lambda *_, _r=rank: (0,) * _r)


def _tile(n, want):
    tn = min(n, want)
    while n % tn:
        tn -= 1
    return tn


def kernel(x, w1, b1, gamma, beta, w2, b2, wp, bp):
    n, ci, ell = x.shape
    co = w1.shape[0]
    has_proj = wp is not None

    w1_k = _conv_weight(w1)
    w2_k = _conv_weight(w2)
    b1_2 = b1.reshape(co, 1)
    b2_2 = (b2 + (bp if has_proj else 0.0)).reshape(co, 1)

    tn1 = _tile(n, _TILE_STATS)
    g1 = n // tn1
    stats = pl.pallas_call(
        _stats_kernel,
        out_shape=jax.ShapeDtypeStruct((g1, co, 128), jnp.float32),
        grid=(g1,),
        in_specs=[pl.BlockSpec((tn1, ci, ell), lambda i: (i, 0, 0)),
                  _const_spec(w1_k.shape), _const_spec(b1_2.shape)],
        out_specs=pl.BlockSpec((1, co, 128), lambda i: (i, 0, 0)),
        compiler_params=pltpu.CompilerParams(
            dimension_semantics=("arbitrary",), vmem_limit_bytes=_VMEM_LIMIT),
    )(x, w1_k, b1_2)

    inv_count = 1.0 / float(n * ell)
    mean = jnp.sum(stats[:, :, 0], axis=0).reshape(co, 1) * inv_count
    var = jnp.sum(stats[:, :, 1], axis=0).reshape(co, 1) * inv_count - mean * mean
    scale = gamma.reshape(co, 1) * lax.rsqrt(var + _BN_EPS)
    shift = beta.reshape(co, 1) - mean * scale

    tn2 = _tile(n, _TILE_APPLY)
    g2 = n // tn2
    x_spec = pl.BlockSpec((tn2, ci, ell), lambda i: (i, 0, 0))
    ins = [x, scale, shift, w1_k, b1_2, w2_k, b2_2]
    in_specs = [x_spec, _const_spec(scale.shape), _const_spec(shift.shape),
                _const_spec(w1_k.shape), _const_spec(b1_2.shape),
                _const_spec(w2_k.shape), _const_spec(b2_2.shape)]
    if has_proj:
        wp_2 = wp[:, :, 0].astype(jnp.bfloat16)
        ins.append(wp_2)
        in_specs.append(_const_spec(wp_2.shape))

    return pl.pallas_call(
        functools.partial(_apply_kernel, has_proj),
        out_shape=jax.ShapeDtypeStruct((n, co, ell), jnp.float32),
        grid=(g2,),
        in_specs=in_specs,
        out_specs=pl.BlockSpec((tn2, co, ell), lambda i: (i, 0, 0)),
        compiler_params=pltpu.CompilerParams(
            dimension_semantics=("arbitrary",), vmem_limit_bytes=_VMEM_LIMIT),
    )(*ins)
